# Initial kernel scaffold; baseline (speedup 1.0000x reference)
import jax
import jax.numpy as jnp
from jax.experimental import pallas as pl


def kernel(graph, features, W1, W2, W3):
    raise NotImplementedError("write your pallas kernel here")


# speedup vs baseline: 3.7482x; 3.7482x over previous
import jax
import jax.numpy as jnp
from jax import lax
from jax.experimental import pallas as pl
from jax.experimental.pallas import tpu as pltpu
from jax.experimental.pallas import tpu_sc as plsc

N = 10000
E = 320000
F = 128
FH = F // 2

NC = 2
NS = 16
NW = NC * NS
CH = 128
NCHUNK = E // CH
CPT = -(-NCHUNK // NS)
DCPT = -(-NCHUNK // NW)
EP = NW * CH * DCPT
NP = 12800
NPA = 10112
RPT = NPA // NS

_mesh = plsc.VectorSubcoreMesh(core_axis_name="c", subcore_axis_name="s",
                               num_cores=NC, num_subcores=NS)


def _deg_body(dst_hbm, ones_hbm, zeros_hbm, deg_hbm,
              idx_v, ones_v, bounce_v, acc_sh):
    c = lax.axis_index("c")
    s = lax.axis_index("s")
    w = c * NS + s
    pltpu.sync_copy(zeros_hbm, bounce_v)
    pltpu.sync_copy(bounce_v, acc_sh.at[pl.ds(s * RPT, RPT)])
    pltpu.sync_copy(ones_hbm, ones_v)
    plsc.subcore_barrier()

    def step(t, carry):
        base = (w * DCPT + t) * CH
        pltpu.sync_copy(dst_hbm.at[pl.ds(base, CH)], idx_v)
        pltpu.sync_copy(ones_v, acc_sh.at[idx_v], add=True)
        return carry

    lax.fori_loop(0, DCPT, step, 0)
    plsc.subcore_barrier()
    pltpu.sync_copy(acc_sh.at[pl.ds(s * RPT, RPT)], bounce_v)
    pltpu.sync_copy(bounce_v, deg_hbm.at[pl.ds(c * NP + s * RPT, RPT)])


def _agg_body(src_hbm, dst_hbm, x_hbm, zeros_hbm, part_hbm,
              src_v, dst_v, rows_v, bounce_v, acc_sh, sem):
    c = lax.axis_index("c")
    s = lax.axis_index("s")
    pltpu.sync_copy(zeros_hbm, bounce_v)
    pltpu.sync_copy(bounce_v, acc_sh.at[pl.ds(s * RPT, RPT)])
    plsc.subcore_barrier()
    coff = c * N

    def step(t, carry):
        base = (s * CPT + t) * CH
        pltpu.sync_copy(src_hbm.at[pl.ds(base, CH)], src_v)
        pltpu.sync_copy(dst_hbm.at[pl.ds(base, CH)], dst_v)
        for j in range(CH // 16):
            sl = pl.ds(j * 16, 16)
            src_v[sl] = src_v[sl] + coff
        pltpu.async_copy(x_hbm.at[src_v], rows_v, sem).wait()
        pltpu.sync_copy(rows_v, acc_sh.at[dst_v], add=True)
        return carry

    lax.fori_loop(0, CPT, step, 0)
    plsc.subcore_barrier()
    pltpu.sync_copy(acc_sh.at[pl.ds(s * RPT, RPT)], bounce_v)
    pltpu.sync_copy(bounce_v, part_hbm.at[pl.ds(c * NP + s * RPT, RPT)])


_sc_params = pltpu.CompilerParams(use_tc_tiling_on_sc=False)

_deg_call = pl.kernel(
    _deg_body,
    out_type=jax.ShapeDtypeStruct((2 * NP, 16), jnp.float32),
    mesh=_mesh,
    compiler_params=_sc_params,
    scratch_types=[
        pltpu.VMEM((CH,), jnp.int32),
        pltpu.VMEM((CH, 16), jnp.float32),
        pltpu.VMEM((RPT, 16), jnp.float32),
        pltpu.VMEM_SHARED((NPA, 16), jnp.float32),
    ],
)

_agg_call = pl.kernel(
    _agg_body,
    out_type=jax.ShapeDtypeStruct((2 * NP, FH), jnp.float32),
    mesh=_mesh,
    compiler_params=_sc_params,
    scratch_types=[
        pltpu.VMEM((CH,), jnp.int32),
        pltpu.VMEM((CH,), jnp.int32),
        pltpu.VMEM((CH, FH), jnp.float32),
        pltpu.VMEM((RPT, FH), jnp.float32),
        pltpu.VMEM_SHARED((NPA, FH), jnp.float32),
        pltpu.SemaphoreType.DMA,
    ],
)


_R = 400
_G = N // _R
_O = NP // _R


def _prep_body(d0_ref, d1_ref, x_ref, dis_ref, s2_ref):
    dsum = d0_ref[...] + d1_ref[...]
    deg = jnp.maximum(dsum[:, 0:1], 1.0)
    dis = lax.rsqrt(deg)
    dis_ref[...] = dis
    s0 = x_ref[...] * dis
    s2_ref[0] = s0[:, :FH]
    s2_ref[1] = s0[:, FH:]


_prep_call = pl.pallas_call(
    _prep_body,
    grid=(_G,),
    in_specs=[
        pl.BlockSpec((_R, 16), lambda i: (i, 0)),
        pl.BlockSpec((_R, 16), lambda i: (i + _O, 0)),
        pl.BlockSpec((_R, F), lambda i: (i, 0)),
    ],
    out_specs=[
        pl.BlockSpec((_R, 1), lambda i: (i, 0)),
        pl.BlockSpec((2, _R, FH), lambda i: (0, i, 0)),
    ],
    out_shape=[
        jax.ShapeDtypeStruct((N, 1), jnp.float32),
        jax.ShapeDtypeStruct((2, N, FH), jnp.float32),
    ],
)


def _layer_body(plo_ref, phi_ref, dis_ref, w_ref, y_ref, snext_ref):
    agg = jnp.concatenate([plo_ref[...], phi_ref[...]], axis=1)
    a = agg * dis_ref[...]
    y = jnp.dot(a, w_ref[...], preferred_element_type=jnp.float32)
    y_ref[...] = y
    s = jnp.maximum(y, 0.0) * dis_ref[...]
    snext_ref[0] = s[:, :FH]
    snext_ref[1] = s[:, FH:]


_layer_call = pl.pallas_call(
    _layer_body,
    grid=(_G,),
    in_specs=[
        pl.BlockSpec((_R, FH), lambda i: (i, 0)),
        pl.BlockSpec((_R, FH), lambda i: (i + _O, 0)),
        pl.BlockSpec((_R, 1), lambda i: (i, 0)),
        pl.BlockSpec((F, F), lambda i: (0, 0)),
    ],
    out_specs=[
        pl.BlockSpec((_R, F), lambda i: (i, 0)),
        pl.BlockSpec((2, _R, FH), lambda i: (0, i, 0)),
    ],
    out_shape=[
        jax.ShapeDtypeStruct((N, F), jnp.float32),
        jax.ShapeDtypeStruct((2, N, FH), jnp.float32),
    ],
)


def kernel(graph, features, W1, W2, W3):
    src = jnp.pad(graph[0], (0, EP - E))
    dst = jnp.pad(graph[1], (0, EP - E), constant_values=N)
    ones_d = jnp.ones((CH, 16), jnp.float32)
    zeros_d = jnp.zeros((RPT, 16), jnp.float32)
    zeros_f = jnp.zeros((RPT, FH), jnp.float32)
    deg = _deg_call(dst, ones_d, zeros_d)
    dis, s2 = _prep_call(deg, deg, features)

    def step(carry, W):
        s2, _ = carry
        part = _agg_call(src, dst, s2.reshape(2 * N, FH), zeros_f)
        y, s2_next = _layer_call(part, part, dis, W)
        return (s2_next, y), None

    y0 = jnp.zeros((N, F), jnp.float32)
    (_, y), _ = lax.scan(step, (s2, y0), jnp.stack([W1, W2, W3]))
    return y


# speedup vs baseline: 4.8781x; 1.3015x over previous
import jax
import jax.numpy as jnp
from jax import lax
from jax.experimental import pallas as pl
from jax.experimental.pallas import tpu as pltpu
from jax.experimental.pallas import tpu_sc as plsc

N = 10000
E = 320000
F = 128
FH = F // 2

NC = 2
NS = 16
NW = NC * NS
CH = 128
NBUF = 4
CPT = 160
NCHUNK = NS * CPT
EP = NCHUNK * CH
DCPT = NCHUNK // NW
NP = 12800
NPA = 10112
RPT = NPA // NS

_mesh = plsc.VectorSubcoreMesh(core_axis_name="c", subcore_axis_name="s",
                               num_cores=NC, num_subcores=NS)


def _deg_body(dst_hbm, ones_hbm, zeros_hbm, deg_hbm,
              idx_v, ones_v, acc_sh):
    c = lax.axis_index("c")
    s = lax.axis_index("s")
    w = c * NS + s
    pltpu.sync_copy(zeros_hbm, acc_sh.at[pl.ds(s * RPT, RPT)])
    pltpu.sync_copy(ones_hbm, ones_v)
    pltpu.sync_copy(dst_hbm.at[pl.ds(w * DCPT, DCPT)], idx_v)
    plsc.subcore_barrier()

    def step(t, carry):
        pltpu.sync_copy(ones_v, acc_sh.at[idx_v.at[t]], add=True)
        return carry

    lax.fori_loop(0, DCPT, step, 0)
    plsc.subcore_barrier()
    pltpu.sync_copy(acc_sh.at[pl.ds(s * RPT, RPT)],
                    deg_hbm.at[pl.ds(c * NP + s * RPT, RPT)])


def _agg_body(src_hbm, dst_hbm, x_hbm, zeros_hbm, part_hbm,
              src_v, dst_v, r0, r1, r2, r3, acc_sh,
              sem0, sem1, sem2, sem3):
    c = lax.axis_index("c")
    s = lax.axis_index("s")
    rows = [r0, r1, r2, r3]
    sems = [sem0, sem1, sem2, sem3]
    pltpu.sync_copy(zeros_hbm, acc_sh.at[pl.ds(s * RPT, RPT)])
    pltpu.sync_copy(src_hbm.at[pl.ds(c * NCHUNK + s * CPT, CPT)], src_v)
    pltpu.sync_copy(dst_hbm.at[pl.ds(s * CPT, CPT)], dst_v)
    plsc.subcore_barrier()

    def gather(t, b):
        return pltpu.make_async_copy(x_hbm.at[src_v.at[t]], rows[b], sems[b])

    for b in range(NBUF):
        gather(b, b).start()

    def step(g, carry):
        for b in range(NBUF):
            t = g * NBUF + b
            gather(t, b).wait()
            pltpu.sync_copy(rows[b], acc_sh.at[dst_v.at[t]], add=True)
            gather(t + NBUF, b).start()
        return carry

    lax.fori_loop(0, CPT // NBUF - 1, step, 0)
    for b in range(NBUF):
        t = CPT - NBUF + b
        gather(t, b).wait()
        pltpu.sync_copy(rows[b], acc_sh.at[dst_v.at[t]], add=True)

    plsc.subcore_barrier()
    pltpu.sync_copy(acc_sh.at[pl.ds(s * RPT, RPT)],
                    part_hbm.at[pl.ds(c * NP + s * RPT, RPT)])


_sc_params = pltpu.CompilerParams(use_tc_tiling_on_sc=False)

_deg_call = pl.kernel(
    _deg_body,
    out_type=jax.ShapeDtypeStruct((2 * NP, 16), jnp.float32),
    mesh=_mesh,
    compiler_params=_sc_params,
    scratch_types=[
        pltpu.VMEM((DCPT, CH), jnp.int32),
        pltpu.VMEM((CH, 16), jnp.float32),
        pltpu.VMEM_SHARED((NPA, 16), jnp.float32),
    ],
)

_agg_call = pl.kernel(
    _agg_body,
    out_type=jax.ShapeDtypeStruct((2 * NP, FH), jnp.float32),
    mesh=_mesh,
    compiler_params=_sc_params,
    scratch_types=[
        pltpu.VMEM((CPT, CH), jnp.int32),
        pltpu.VMEM((CPT, CH), jnp.int32),
        pltpu.VMEM((CH, FH), jnp.float32),
        pltpu.VMEM((CH, FH), jnp.float32),
        pltpu.VMEM((CH, FH), jnp.float32),
        pltpu.VMEM((CH, FH), jnp.float32),
        pltpu.VMEM_SHARED((NPA, FH), jnp.float32),
        pltpu.SemaphoreType.DMA,
        pltpu.SemaphoreType.DMA,
        pltpu.SemaphoreType.DMA,
        pltpu.SemaphoreType.DMA,
    ],
)


_R = 400
_G = N // _R
_O = NP // _R


def _prep_body(d0_ref, d1_ref, x_ref, dis_ref, s2_ref):
    dsum = d0_ref[...] + d1_ref[...]
    deg = jnp.maximum(dsum[:, 0:1], 1.0)
    dis = lax.rsqrt(deg)
    dis_ref[...] = dis
    s0 = x_ref[...] * dis
    s2_ref[0] = s0[:, :FH]
    s2_ref[1] = s0[:, FH:]


_prep_call = pl.pallas_call(
    _prep_body,
    grid=(_G,),
    in_specs=[
        pl.BlockSpec((_R, 16), lambda i: (i, 0)),
        pl.BlockSpec((_R, 16), lambda i: (i + _O, 0)),
        pl.BlockSpec((_R, F), lambda i: (i, 0)),
    ],
    out_specs=[
        pl.BlockSpec((_R, 1), lambda i: (i, 0)),
        pl.BlockSpec((2, _R, FH), lambda i: (0, i, 0)),
    ],
    out_shape=[
        jax.ShapeDtypeStruct((N, 1), jnp.float32),
        jax.ShapeDtypeStruct((2, N, FH), jnp.float32),
    ],
)


def _layer_body(plo_ref, phi_ref, dis_ref, w_ref, y_ref, snext_ref):
    agg = jnp.concatenate([plo_ref[...], phi_ref[...]], axis=1)
    a = agg * dis_ref[...]
    y = jnp.dot(a, w_ref[...], preferred_element_type=jnp.float32)
    y_ref[...] = y
    s = jnp.maximum(y, 0.0) * dis_ref[...]
    snext_ref[0] = s[:, :FH]
    snext_ref[1] = s[:, FH:]


_layer_call = pl.pallas_call(
    _layer_body,
    grid=(_G,),
    in_specs=[
        pl.BlockSpec((_R, FH), lambda i: (i, 0)),
        pl.BlockSpec((_R, FH), lambda i: (i + _O, 0)),
        pl.BlockSpec((_R, 1), lambda i: (i, 0)),
        pl.BlockSpec((F, F), lambda i: (0, 0)),
    ],
    out_specs=[
        pl.BlockSpec((_R, F), lambda i: (i, 0)),
        pl.BlockSpec((2, _R, FH), lambda i: (0, i, 0)),
    ],
    out_shape=[
        jax.ShapeDtypeStruct((N, F), jnp.float32),
        jax.ShapeDtypeStruct((2, N, FH), jnp.float32),
    ],
)


def kernel(graph, features, W1, W2, W3):
    srcp = jnp.pad(graph[0], (0, EP - E))
    dstp = jnp.pad(graph[1], (0, EP - E), constant_values=N)
    src2 = jnp.concatenate([srcp, srcp + N]).reshape(2 * NCHUNK, CH)
    dst2 = dstp.reshape(NCHUNK, CH)
    ones_d = jnp.ones((CH, 16), jnp.float32)
    zeros_d = jnp.zeros((RPT, 16), jnp.float32)
    zeros_f = jnp.zeros((RPT, FH), jnp.float32)
    deg = _deg_call(dst2, ones_d, zeros_d)
    dis, s2 = _prep_call(deg, deg, features)

    def step(carry, W):
        s2, _ = carry
        part = _agg_call(src2, dst2, s2.reshape(2 * N, FH), zeros_f)
        y, s2_next = _layer_call(part, part, dis, W)
        return (s2_next, y), None

    y0 = jnp.zeros((N, F), jnp.float32)
    (_, y), _ = lax.scan(step, (s2, y0), jnp.stack([W1, W2, W3]))
    return y
